# Initial kernel scaffold; baseline (speedup 1.0000x reference)
import jax
import jax.numpy as jnp
from jax.experimental import pallas as pl


def kernel(input_data, prompt_keys, prompt_values, top_k):
    raise NotImplementedError("write your pallas kernel here")


# speedup vs baseline: 2.1607x; 2.1607x over previous
import functools

import jax
import jax.numpy as jnp
from jax import lax
from jax.experimental import pallas as pl
from jax.experimental.pallas import tpu as pltpu

_B = 8192
_D = 1024
_P = 64
_K = 8
_BLK = 128
_EPS = 1e-12


def _body(x_ref, k_ref, v_ref, out_ref, loss_ref, idx_ref):
    i = pl.program_id(0)
    nprog = pl.num_programs(0)

    x = x_ref[...]
    xn = x / jnp.maximum(jnp.sqrt(jnp.sum(x * x, axis=1, keepdims=True)), _EPS)
    k = k_ref[...]
    kn = k / jnp.maximum(jnp.sqrt(jnp.sum(k * k, axis=1, keepdims=True)), _EPS)

    s = lax.dot_general(xn, kn, (((1,), (1,)), ((), ())),
                        preferred_element_type=jnp.float32)
    m = jnp.max(s, axis=1, keepdims=True)
    e = jnp.exp(s - m)
    p = e / jnp.sum(e, axis=1, keepdims=True)

    cols = lax.broadcasted_iota(jnp.int32, (_BLK, _P), 1)
    v = v_ref[...]

    work = p
    val_sum = jnp.zeros((), jnp.float32)
    for j in range(_K):
        mx = jnp.max(work, axis=1, keepdims=True)
        amx = jnp.min(jnp.where(work == mx, cols, _P), axis=1,
                      keepdims=True)
        oh = (cols == amx).astype(jnp.float32)
        sel = lax.dot_general(oh, v, (((1,), (0,)), ((), ())),
                              preferred_element_type=jnp.float32)
        out_ref[:, j, :] = sel
        idx_ref[:, j] = amx[:, 0]
        val_sum = val_sum + jnp.sum(mx)
        work = jnp.where(cols == amx, -1.0, work)

    @pl.when(i == 0)
    def _():
        loss_ref[0, 0] = 0.0

    loss_ref[0, 0] += val_sum

    @pl.when(i == nprog - 1)
    def _():
        loss_ref[0, 0] = loss_ref[0, 0] * (-1.0 / _B)


@functools.partial(jax.jit, static_argnames=())
def _run(input_data, prompt_keys, prompt_values):
    grid = _B // _BLK
    sel, loss, idxs = pl.pallas_call(
        _body,
        grid=(grid,),
        in_specs=[
            pl.BlockSpec((_BLK, _D), lambda i: (i, 0)),
            pl.BlockSpec((_P, _D), lambda i: (0, 0)),
            pl.BlockSpec((_P, _D), lambda i: (0, 0)),
        ],
        out_specs=[
            pl.BlockSpec((_BLK, _K, _D), lambda i: (i, 0, 0)),
            pl.BlockSpec((1, 1), lambda i: (0, 0),
                         memory_space=pltpu.SMEM),
            pl.BlockSpec((_BLK, _K), lambda i: (i, 0)),
        ],
        out_shape=[
            jax.ShapeDtypeStruct((_B, _K, _D), jnp.float32),
            jax.ShapeDtypeStruct((1, 1), jnp.float32),
            jax.ShapeDtypeStruct((_B, _K), jnp.int32),
        ],
        compiler_params=pltpu.CompilerParams(
            dimension_semantics=("arbitrary",),
        ),
    )(input_data, prompt_keys, prompt_values)
    return sel, loss[0, 0], idxs


def kernel(input_data, prompt_keys, prompt_values, top_k):
    del top_k
    return _run(input_data, prompt_keys, prompt_values)


# speedup vs baseline: 2.8819x; 1.3338x over previous
import functools

import jax
import jax.numpy as jnp
from jax import lax
from jax.experimental import pallas as pl
from jax.experimental.pallas import tpu as pltpu

_B = 8192
_D = 1024
_P = 64
_K = 8
_BLK = 256
_EPS = 1e-12


def _body(x_ref, k_ref, v_ref, out_ref, loss_ref, idx_ref):
    i = pl.program_id(0)
    nprog = pl.num_programs(0)

    x = x_ref[...]
    xn = x / jnp.maximum(jnp.sqrt(jnp.sum(x * x, axis=1, keepdims=True)), _EPS)
    k = k_ref[...]
    kn = k / jnp.maximum(jnp.sqrt(jnp.sum(k * k, axis=1, keepdims=True)), _EPS)

    s = lax.dot_general(xn, kn, (((1,), (1,)), ((), ())),
                        preferred_element_type=jnp.float32)
    m = jnp.max(s, axis=1, keepdims=True)
    e = jnp.exp(s - m)
    p = e / jnp.sum(e, axis=1, keepdims=True)

    cols = lax.broadcasted_iota(jnp.int32, (_BLK, _P), 1)
    v = v_ref[...]

    work = p
    val_sum = jnp.zeros((), jnp.float32)
    for j in range(_K):
        mx = jnp.max(work, axis=1, keepdims=True)
        amx = jnp.min(jnp.where(work == mx, cols, _P), axis=1,
                      keepdims=True)
        oh = (cols == amx).astype(jnp.float32)
        sel = lax.dot_general(oh, v, (((1,), (0,)), ((), ())),
                              preferred_element_type=jnp.float32)
        out_ref[:, j, :] = sel
        idx_ref[:, j] = amx[:, 0]
        val_sum = val_sum + jnp.sum(mx)
        work = jnp.where(cols == amx, -1.0, work)

    @pl.when(i == 0)
    def _():
        loss_ref[0, 0] = 0.0

    loss_ref[0, 0] += val_sum

    @pl.when(i == nprog - 1)
    def _():
        loss_ref[0, 0] = loss_ref[0, 0] * (-1.0 / _B)


@functools.partial(jax.jit, static_argnames=())
def _run(input_data, prompt_keys, prompt_values):
    grid = _B // _BLK
    sel, loss, idxs = pl.pallas_call(
        _body,
        grid=(grid,),
        in_specs=[
            pl.BlockSpec((_BLK, _D), lambda i: (i, 0)),
            pl.BlockSpec((_P, _D), lambda i: (0, 0)),
            pl.BlockSpec((_P, _D), lambda i: (0, 0)),
        ],
        out_specs=[
            pl.BlockSpec((_BLK, _K, _D), lambda i: (i, 0, 0)),
            pl.BlockSpec((1, 1), lambda i: (0, 0),
                         memory_space=pltpu.SMEM),
            pl.BlockSpec((_BLK, _K), lambda i: (i, 0)),
        ],
        out_shape=[
            jax.ShapeDtypeStruct((_B, _K, _D), jnp.float32),
            jax.ShapeDtypeStruct((1, 1), jnp.float32),
            jax.ShapeDtypeStruct((_B, _K), jnp.int32),
        ],
        compiler_params=pltpu.CompilerParams(
            dimension_semantics=("arbitrary",),
        ),
    )(input_data, prompt_keys, prompt_values)
    return sel, loss[0, 0], idxs


def kernel(input_data, prompt_keys, prompt_values, top_k):
    del top_k
    return _run(input_data, prompt_keys, prompt_values)


# speedup vs baseline: 3.0299x; 1.0514x over previous
import functools

import jax
import jax.numpy as jnp
from jax import lax
from jax.experimental import pallas as pl
from jax.experimental.pallas import tpu as pltpu

_B = 8192
_D = 1024
_P = 64
_K = 8
_BLK = 512
_EPS = 1e-12


def _body(x_ref, k_ref, v_ref, out_ref, loss_ref, idx_ref):
    i = pl.program_id(0)
    nprog = pl.num_programs(0)

    x = x_ref[...]
    xn = x / jnp.maximum(jnp.sqrt(jnp.sum(x * x, axis=1, keepdims=True)), _EPS)
    k = k_ref[...]
    kn = k / jnp.maximum(jnp.sqrt(jnp.sum(k * k, axis=1, keepdims=True)), _EPS)

    s = lax.dot_general(xn, kn, (((1,), (1,)), ((), ())),
                        preferred_element_type=jnp.float32)
    m = jnp.max(s, axis=1, keepdims=True)
    e = jnp.exp(s - m)
    p = e / jnp.sum(e, axis=1, keepdims=True)

    cols = lax.broadcasted_iota(jnp.int32, (_BLK, _P), 1)
    v = v_ref[...]

    work = p
    val_sum = jnp.zeros((), jnp.float32)
    for j in range(_K):
        mx = jnp.max(work, axis=1, keepdims=True)
        amx = jnp.min(jnp.where(work == mx, cols, _P), axis=1,
                      keepdims=True)
        oh = (cols == amx).astype(jnp.float32)
        sel = lax.dot_general(oh, v, (((1,), (0,)), ((), ())),
                              preferred_element_type=jnp.float32)
        out_ref[:, j, :] = sel
        idx_ref[:, j] = amx[:, 0]
        val_sum = val_sum + jnp.sum(mx)
        work = jnp.where(cols == amx, -1.0, work)

    @pl.when(i == 0)
    def _():
        loss_ref[0, 0] = 0.0

    loss_ref[0, 0] += val_sum

    @pl.when(i == nprog - 1)
    def _():
        loss_ref[0, 0] = loss_ref[0, 0] * (-1.0 / _B)


@functools.partial(jax.jit, static_argnames=())
def _run(input_data, prompt_keys, prompt_values):
    grid = _B // _BLK
    sel, loss, idxs = pl.pallas_call(
        _body,
        grid=(grid,),
        in_specs=[
            pl.BlockSpec((_BLK, _D), lambda i: (i, 0)),
            pl.BlockSpec((_P, _D), lambda i: (0, 0)),
            pl.BlockSpec((_P, _D), lambda i: (0, 0)),
        ],
        out_specs=[
            pl.BlockSpec((_BLK, _K, _D), lambda i: (i, 0, 0)),
            pl.BlockSpec((1, 1), lambda i: (0, 0),
                         memory_space=pltpu.SMEM),
            pl.BlockSpec((_BLK, _K), lambda i: (i, 0)),
        ],
        out_shape=[
            jax.ShapeDtypeStruct((_B, _K, _D), jnp.float32),
            jax.ShapeDtypeStruct((1, 1), jnp.float32),
            jax.ShapeDtypeStruct((_B, _K), jnp.int32),
        ],
        compiler_params=pltpu.CompilerParams(
            dimension_semantics=("arbitrary",),
        ),
    )(input_data, prompt_keys, prompt_values)
    return sel, loss[0, 0], idxs


def kernel(input_data, prompt_keys, prompt_values, top_k):
    del top_k
    return _run(input_data, prompt_keys, prompt_values)
